# initial kernel scaffold (unmeasured)
import jax
import jax.numpy as jnp
from jax import lax
from jax.experimental import pallas as pl
from jax.experimental.pallas import tpu as pltpu

M = 4096
D = 4096
KQ = 4096
HALF = M // 2
QTR = M // 4

MESH = pl.DeviceIdType.MESH


def _allreduce_body(p_ref, out_ref, recv1, stage1, recv2,
                    send_sems, recv_sems, local_sems):
    my_x = lax.axis_index("x")
    my_y = lax.axis_index("y")
    x_partner = (1 - my_x, my_y)
    y_partner = (my_x, 1 - my_y)

    barrier = pltpu.get_barrier_semaphore()
    pl.semaphore_signal(barrier, inc=1, device_id=x_partner,
                        device_id_type=MESH)
    pl.semaphore_signal(barrier, inc=1, device_id=y_partner,
                        device_id_type=MESH)
    pl.semaphore_wait(barrier, 2)

    my_half = my_x * HALF
    other_half = (1 - my_x) * HALF

    rdma1 = pltpu.make_async_remote_copy(
        src_ref=p_ref.at[pl.ds(other_half, HALF), :],
        dst_ref=recv1,
        send_sem=send_sems.at[0],
        recv_sem=recv_sems.at[0],
        device_id=x_partner,
        device_id_type=MESH,
    )
    rdma1.start()
    cp1 = pltpu.make_async_copy(
        p_ref.at[pl.ds(my_half, HALF), :], stage1, local_sems.at[0])
    cp1.start()
    cp1.wait()
    rdma1.wait()
    recv1[...] = recv1[...] + stage1[...]

    rdma2 = pltpu.make_async_remote_copy(
        src_ref=recv1.at[pl.ds((1 - my_y) * QTR, QTR), :],
        dst_ref=recv2,
        send_sem=send_sems.at[1],
        recv_sem=recv_sems.at[1],
        device_id=y_partner,
        device_id_type=MESH,
    )
    rdma2.start()
    rdma2.wait()
    recv2[...] = recv2[...] + recv1[pl.ds(my_y * QTR, QTR), :]

    gq = my_x * HALF + my_y * QTR

    cp2 = pltpu.make_async_copy(
        recv2, out_ref.at[pl.ds(gq, QTR), :], local_sems.at[1])
    cp2.start()

    rdma3 = pltpu.make_async_remote_copy(
        src_ref=recv2,
        dst_ref=out_ref.at[pl.ds(gq, QTR), :],
        send_sem=send_sems.at[2],
        recv_sem=recv_sems.at[2],
        device_id=y_partner,
        device_id_type=MESH,
    )
    rdma3.start()
    cp2.wait()
    rdma3.wait()

    rdma4 = pltpu.make_async_remote_copy(
        src_ref=out_ref.at[pl.ds(my_half, HALF), :],
        dst_ref=out_ref.at[pl.ds(my_half, HALF), :],
        send_sem=send_sems.at[3],
        recv_sem=recv_sems.at[3],
        device_id=x_partner,
        device_id_type=MESH,
    )
    rdma4.start()
    rdma4.wait()


def _allreduce_bf16(p):
    return pl.pallas_call(
        _allreduce_body,
        out_shape=jax.ShapeDtypeStruct((M, D), jnp.bfloat16),
        in_specs=[pl.BlockSpec(memory_space=pltpu.ANY)],
        out_specs=pl.BlockSpec(memory_space=pltpu.ANY),
        scratch_shapes=[
            pltpu.VMEM((HALF, D), jnp.bfloat16),
            pltpu.VMEM((HALF, D), jnp.bfloat16),
            pltpu.VMEM((QTR, D), jnp.bfloat16),
            pltpu.SemaphoreType.DMA((4,)),
            pltpu.SemaphoreType.DMA((4,)),
            pltpu.SemaphoreType.DMA((2,)),
        ],
        compiler_params=pltpu.CompilerParams(collective_id=0),
    )(p)


def kernel(dy, W):
    my_y = lax.axis_index("y")
    dy_q = lax.dynamic_slice_in_dim(dy, my_y * KQ, KQ, axis=1)
    W_q = lax.dynamic_slice_in_dim(W, my_y * KQ, KQ, axis=1)
    partial = lax.dot_general(
        dy_q.astype(jnp.bfloat16), W_q.astype(jnp.bfloat16),
        (((1,), (1,)), ((), ())),
        preferred_element_type=jnp.bfloat16,
    )
    return _allreduce_bf16(partial).astype(jnp.float32)


# baseline (device time: 843809 ns/iter reference)
import jax
import jax.numpy as jnp
from jax import lax
from jax.experimental import pallas as pl
from jax.experimental.pallas import tpu as pltpu

M = 4096
D = 4096
KQ = 4096
HALF = M // 2
QTR = M // 4

MESH = pl.DeviceIdType.MESH


def _allreduce_body(p_ref, out_ref, recv1, stage1, recv2,
                    send_sems, recv_sems, local_sems):
    my_x = lax.axis_index("x")
    my_y = lax.axis_index("y")
    x_partner = (1 - my_x, my_y)
    y_partner = (my_x, 1 - my_y)

    barrier = pltpu.get_barrier_semaphore()
    pl.semaphore_signal(barrier, inc=1, device_id=x_partner,
                        device_id_type=MESH)
    pl.semaphore_signal(barrier, inc=1, device_id=y_partner,
                        device_id_type=MESH)
    pl.semaphore_wait(barrier, 2)

    my_half = my_x * HALF
    other_half = (1 - my_x) * HALF

    rdma1 = pltpu.make_async_remote_copy(
        src_ref=p_ref.at[pl.ds(other_half, HALF), :],
        dst_ref=recv1,
        send_sem=send_sems.at[0],
        recv_sem=recv_sems.at[0],
        device_id=x_partner,
        device_id_type=MESH,
    )
    rdma1.start()
    cp1 = pltpu.make_async_copy(
        p_ref.at[pl.ds(my_half, HALF), :], stage1, local_sems.at[0])
    cp1.start()
    cp1.wait()
    rdma1.wait()
    recv1[...] = recv1[...] + stage1[...]

    rdma2 = pltpu.make_async_remote_copy(
        src_ref=recv1.at[pl.ds((1 - my_y) * QTR, QTR), :],
        dst_ref=recv2,
        send_sem=send_sems.at[1],
        recv_sem=recv_sems.at[1],
        device_id=y_partner,
        device_id_type=MESH,
    )
    rdma2.start()
    rdma2.wait()
    recv2[...] = recv2[...] + recv1[pl.ds(my_y * QTR, QTR), :]

    gq = my_x * HALF + my_y * QTR

    cp2 = pltpu.make_async_copy(
        recv2, out_ref.at[pl.ds(gq, QTR), :], local_sems.at[1])
    cp2.start()

    rdma3 = pltpu.make_async_remote_copy(
        src_ref=recv2,
        dst_ref=out_ref.at[pl.ds(gq, QTR), :],
        send_sem=send_sems.at[2],
        recv_sem=recv_sems.at[2],
        device_id=y_partner,
        device_id_type=MESH,
    )
    rdma3.start()
    cp2.wait()
    rdma3.wait()

    rdma4 = pltpu.make_async_remote_copy(
        src_ref=out_ref.at[pl.ds(my_half, HALF), :],
        dst_ref=out_ref.at[pl.ds(my_half, HALF), :],
        send_sem=send_sems.at[3],
        recv_sem=recv_sems.at[3],
        device_id=x_partner,
        device_id_type=MESH,
    )
    rdma4.start()
    rdma4.wait()


def _allreduce_bf16(p):
    return pl.pallas_call(
        _allreduce_body,
        out_shape=jax.ShapeDtypeStruct((M, D), jnp.bfloat16),
        in_specs=[pl.BlockSpec(memory_space=pl.ANY)],
        out_specs=pl.BlockSpec(memory_space=pl.ANY),
        scratch_shapes=[
            pltpu.VMEM((HALF, D), jnp.bfloat16),
            pltpu.VMEM((HALF, D), jnp.bfloat16),
            pltpu.VMEM((QTR, D), jnp.bfloat16),
            pltpu.SemaphoreType.DMA((4,)),
            pltpu.SemaphoreType.DMA((4,)),
            pltpu.SemaphoreType.DMA((2,)),
        ],
        compiler_params=pltpu.CompilerParams(
            collective_id=0,
            vmem_limit_bytes=60 * 1024 * 1024,
        ),
    )(p)


def kernel(dy, W):
    my_y = lax.axis_index("y")
    dy_q = lax.dynamic_slice_in_dim(dy, my_y * KQ, KQ, axis=1)
    W_q = lax.dynamic_slice_in_dim(W, my_y * KQ, KQ, axis=1)
    partial = lax.dot_general(
        dy_q.astype(jnp.bfloat16), W_q.astype(jnp.bfloat16),
        (((1,), (1,)), ((), ())),
        preferred_element_type=jnp.bfloat16,
    )
    return _allreduce_bf16(partial).astype(jnp.float32)


# device time: 573990 ns/iter; 1.4701x vs baseline; 1.4701x over previous
import jax
import jax.numpy as jnp
from jax import lax
from jax.experimental import pallas as pl
from jax.experimental.pallas import tpu as pltpu

M = 4096
D = 4096
KQ = 4096
HALF = M // 2
QTR = M // 4
CH = D // 2

MESH = pl.DeviceIdType.MESH


def _allreduce_body(p_ref, out_ref,
                    recv_a1, stage_a, recv_a2,
                    recv_b1, stage_b, recv_b2,
                    send_sems, recv_sems, local_sems):
    mx = lax.axis_index("x")
    my = lax.axis_index("y")
    x_partner = (1 - mx, my)
    y_partner = (mx, 1 - my)

    barrier = pltpu.get_barrier_semaphore()
    pl.semaphore_signal(barrier, inc=1, device_id=x_partner,
                        device_id_type=MESH)
    pl.semaphore_signal(barrier, inc=1, device_id=y_partner,
                        device_id_type=MESH)
    pl.semaphore_wait(barrier, 2)

    rdma_a1 = pltpu.make_async_remote_copy(
        src_ref=p_ref.at[pl.ds((1 - mx) * HALF, HALF), pl.ds(0, CH)],
        dst_ref=recv_a1,
        send_sem=send_sems.at[0], recv_sem=recv_sems.at[0],
        device_id=x_partner, device_id_type=MESH,
    )
    rdma_b1 = pltpu.make_async_remote_copy(
        src_ref=p_ref.at[pl.ds((1 - my) * HALF, HALF), pl.ds(CH, CH)],
        dst_ref=recv_b1,
        send_sem=send_sems.at[4], recv_sem=recv_sems.at[4],
        device_id=y_partner, device_id_type=MESH,
    )
    rdma_a1.start()
    rdma_b1.start()
    cp_a = pltpu.make_async_copy(
        p_ref.at[pl.ds(mx * HALF, HALF), pl.ds(0, CH)],
        stage_a, local_sems.at[0])
    cp_b = pltpu.make_async_copy(
        p_ref.at[pl.ds(my * HALF, HALF), pl.ds(CH, CH)],
        stage_b, local_sems.at[1])
    cp_a.start()
    cp_b.start()

    cp_a.wait()
    rdma_a1.wait()
    recv_a1[...] = recv_a1[...] + stage_a[...]

    rdma_a2 = pltpu.make_async_remote_copy(
        src_ref=recv_a1.at[pl.ds((1 - my) * QTR, QTR), :],
        dst_ref=recv_a2,
        send_sem=send_sems.at[1], recv_sem=recv_sems.at[1],
        device_id=y_partner, device_id_type=MESH,
    )
    rdma_a2.start()

    cp_b.wait()
    rdma_b1.wait()
    recv_b1[...] = recv_b1[...] + stage_b[...]
    rdma_b2 = pltpu.make_async_remote_copy(
        src_ref=recv_b1.at[pl.ds((1 - mx) * QTR, QTR), :],
        dst_ref=recv_b2,
        send_sem=send_sems.at[5], recv_sem=recv_sems.at[5],
        device_id=x_partner, device_id_type=MESH,
    )
    rdma_b2.start()

    gq_a = mx * HALF + my * QTR
    gq_b = my * HALF + mx * QTR

    rdma_a2.wait()
    recv_a2[...] = recv_a2[...] + recv_a1[pl.ds(my * QTR, QTR), :]
    cp_a2 = pltpu.make_async_copy(
        recv_a2, out_ref.at[pl.ds(gq_a, QTR), pl.ds(0, CH)],
        local_sems.at[2])
    cp_a2.start()
    rdma_a3 = pltpu.make_async_remote_copy(
        src_ref=recv_a2,
        dst_ref=out_ref.at[pl.ds(gq_a, QTR), pl.ds(0, CH)],
        send_sem=send_sems.at[2], recv_sem=recv_sems.at[2],
        device_id=y_partner, device_id_type=MESH,
    )
    rdma_a3.start()

    rdma_b2.wait()
    recv_b2[...] = recv_b2[...] + recv_b1[pl.ds(mx * QTR, QTR), :]
    cp_b2 = pltpu.make_async_copy(
        recv_b2, out_ref.at[pl.ds(gq_b, QTR), pl.ds(CH, CH)],
        local_sems.at[3])
    cp_b2.start()
    rdma_b3 = pltpu.make_async_remote_copy(
        src_ref=recv_b2,
        dst_ref=out_ref.at[pl.ds(gq_b, QTR), pl.ds(CH, CH)],
        send_sem=send_sems.at[6], recv_sem=recv_sems.at[6],
        device_id=x_partner, device_id_type=MESH,
    )
    rdma_b3.start()

    cp_a2.wait()
    rdma_a3.wait()
    rdma_a4 = pltpu.make_async_remote_copy(
        src_ref=out_ref.at[pl.ds(mx * HALF, HALF), pl.ds(0, CH)],
        dst_ref=out_ref.at[pl.ds(mx * HALF, HALF), pl.ds(0, CH)],
        send_sem=send_sems.at[3], recv_sem=recv_sems.at[3],
        device_id=x_partner, device_id_type=MESH,
    )
    rdma_a4.start()

    cp_b2.wait()
    rdma_b3.wait()
    rdma_b4 = pltpu.make_async_remote_copy(
        src_ref=out_ref.at[pl.ds(my * HALF, HALF), pl.ds(CH, CH)],
        dst_ref=out_ref.at[pl.ds(my * HALF, HALF), pl.ds(CH, CH)],
        send_sem=send_sems.at[7], recv_sem=recv_sems.at[7],
        device_id=y_partner, device_id_type=MESH,
    )
    rdma_b4.start()

    rdma_a4.wait()
    rdma_b4.wait()


def _allreduce_bf16(p):
    return pl.pallas_call(
        _allreduce_body,
        out_shape=jax.ShapeDtypeStruct((M, D), jnp.bfloat16),
        in_specs=[pl.BlockSpec(memory_space=pl.ANY)],
        out_specs=pl.BlockSpec(memory_space=pl.ANY),
        scratch_shapes=[
            pltpu.VMEM((HALF, CH), jnp.bfloat16),
            pltpu.VMEM((HALF, CH), jnp.bfloat16),
            pltpu.VMEM((QTR, CH), jnp.bfloat16),
            pltpu.VMEM((HALF, CH), jnp.bfloat16),
            pltpu.VMEM((HALF, CH), jnp.bfloat16),
            pltpu.VMEM((QTR, CH), jnp.bfloat16),
            pltpu.SemaphoreType.DMA((8,)),
            pltpu.SemaphoreType.DMA((8,)),
            pltpu.SemaphoreType.DMA((4,)),
        ],
        compiler_params=pltpu.CompilerParams(
            collective_id=0,
            vmem_limit_bytes=60 * 1024 * 1024,
        ),
    )(p)


def kernel(dy, W):
    my_y = lax.axis_index("y")
    dy_q = lax.dynamic_slice_in_dim(dy, my_y * KQ, KQ, axis=1)
    W_q = lax.dynamic_slice_in_dim(W, my_y * KQ, KQ, axis=1)
    partial = lax.dot_general(
        dy_q.astype(jnp.bfloat16), W_q.astype(jnp.bfloat16),
        (((1,), (1,)), ((), ())),
        preferred_element_type=jnp.bfloat16,
    )
    return _allreduce_bf16(partial).astype(jnp.float32)


# device time: 542773 ns/iter; 1.5546x vs baseline; 1.0575x over previous
import jax
import jax.numpy as jnp
from jax import lax
from jax.experimental import pallas as pl
from jax.experimental.pallas import tpu as pltpu

M = 4096
D = 4096
KQ = 4096
HALF = M // 2
QTR = M // 4
CH = D // 2

MESH = pl.DeviceIdType.MESH


def _allreduce_body(p_ref, out_ref,
                    recv_a1, stage_a, recv_a2,
                    recv_b1, stage_b, recv_b2,
                    send_sems, recv_sems, local_sems):
    mx = lax.axis_index("x")
    my = lax.axis_index("y")
    x_partner = (1 - mx, my)
    y_partner = (mx, 1 - my)

    barrier = pltpu.get_barrier_semaphore()
    pl.semaphore_signal(barrier, inc=1, device_id=x_partner,
                        device_id_type=MESH)
    pl.semaphore_signal(barrier, inc=1, device_id=y_partner,
                        device_id_type=MESH)
    pl.semaphore_wait(barrier, 2)

    rdma_a1 = pltpu.make_async_remote_copy(
        src_ref=p_ref.at[pl.ds((1 - mx) * HALF, HALF), pl.ds(0, CH)],
        dst_ref=recv_a1,
        send_sem=send_sems.at[0], recv_sem=recv_sems.at[0],
        device_id=x_partner, device_id_type=MESH,
    )
    rdma_b1 = pltpu.make_async_remote_copy(
        src_ref=p_ref.at[pl.ds((1 - my) * HALF, HALF), pl.ds(CH, CH)],
        dst_ref=recv_b1,
        send_sem=send_sems.at[4], recv_sem=recv_sems.at[4],
        device_id=y_partner, device_id_type=MESH,
    )
    rdma_a1.start()
    rdma_b1.start()
    cp_a = pltpu.make_async_copy(
        p_ref.at[pl.ds(mx * HALF, HALF), pl.ds(0, CH)],
        stage_a, local_sems.at[0])
    cp_b = pltpu.make_async_copy(
        p_ref.at[pl.ds(my * HALF, HALF), pl.ds(CH, CH)],
        stage_b, local_sems.at[1])
    cp_a.start()
    cp_b.start()

    cp_a.wait()
    rdma_a1.wait()
    recv_a1[...] = recv_a1[...] + stage_a[...]

    rdma_a2 = pltpu.make_async_remote_copy(
        src_ref=recv_a1.at[pl.ds((1 - my) * QTR, QTR), :],
        dst_ref=recv_a2,
        send_sem=send_sems.at[1], recv_sem=recv_sems.at[1],
        device_id=y_partner, device_id_type=MESH,
    )
    rdma_a2.start()

    cp_b.wait()
    rdma_b1.wait()
    recv_b1[...] = recv_b1[...] + stage_b[...]
    rdma_b2 = pltpu.make_async_remote_copy(
        src_ref=recv_b1.at[pl.ds((1 - mx) * QTR, QTR), :],
        dst_ref=recv_b2,
        send_sem=send_sems.at[5], recv_sem=recv_sems.at[5],
        device_id=x_partner, device_id_type=MESH,
    )
    rdma_b2.start()

    gq_a = mx * HALF + my * QTR
    gq_b = my * HALF + mx * QTR

    rdma_a2.wait()
    recv_a2[...] = recv_a2[...] + recv_a1[pl.ds(my * QTR, QTR), :]
    cp_a2 = pltpu.make_async_copy(
        recv_a2, out_ref.at[pl.ds(gq_a, QTR), pl.ds(0, CH)],
        local_sems.at[2])
    cp_a2.start()
    rdma_a3 = pltpu.make_async_remote_copy(
        src_ref=recv_a2,
        dst_ref=out_ref.at[pl.ds(gq_a, QTR), pl.ds(0, CH)],
        send_sem=send_sems.at[2], recv_sem=recv_sems.at[2],
        device_id=y_partner, device_id_type=MESH,
    )
    rdma_a3.start()

    rdma_b2.wait()
    recv_b2[...] = recv_b2[...] + recv_b1[pl.ds(mx * QTR, QTR), :]
    cp_b2 = pltpu.make_async_copy(
        recv_b2, out_ref.at[pl.ds(gq_b, QTR), pl.ds(CH, CH)],
        local_sems.at[3])
    cp_b2.start()
    rdma_b3 = pltpu.make_async_remote_copy(
        src_ref=recv_b2,
        dst_ref=out_ref.at[pl.ds(gq_b, QTR), pl.ds(CH, CH)],
        send_sem=send_sems.at[6], recv_sem=recv_sems.at[6],
        device_id=x_partner, device_id_type=MESH,
    )
    rdma_b3.start()

    cp_a2.wait()
    rdma_a3.wait()
    rdma_a4 = pltpu.make_async_remote_copy(
        src_ref=out_ref.at[pl.ds(mx * HALF, HALF), pl.ds(0, CH)],
        dst_ref=out_ref.at[pl.ds(mx * HALF, HALF), pl.ds(0, CH)],
        send_sem=send_sems.at[3], recv_sem=recv_sems.at[3],
        device_id=x_partner, device_id_type=MESH,
    )
    rdma_a4.start()

    cp_b2.wait()
    rdma_b3.wait()
    rdma_b4 = pltpu.make_async_remote_copy(
        src_ref=out_ref.at[pl.ds(my * HALF, HALF), pl.ds(CH, CH)],
        dst_ref=out_ref.at[pl.ds(my * HALF, HALF), pl.ds(CH, CH)],
        send_sem=send_sems.at[7], recv_sem=recv_sems.at[7],
        device_id=y_partner, device_id_type=MESH,
    )
    rdma_b4.start()

    rdma_a4.wait()
    rdma_b4.wait()


def _allreduce_bf16(p):
    return pl.pallas_call(
        _allreduce_body,
        out_shape=jax.ShapeDtypeStruct((M, D), jnp.bfloat16),
        in_specs=[pl.BlockSpec(memory_space=pl.ANY)],
        out_specs=pl.BlockSpec(memory_space=pl.ANY),
        scratch_shapes=[
            pltpu.VMEM((HALF, CH), jnp.bfloat16),
            pltpu.VMEM((HALF, CH), jnp.bfloat16),
            pltpu.VMEM((QTR, CH), jnp.bfloat16),
            pltpu.VMEM((HALF, CH), jnp.bfloat16),
            pltpu.VMEM((HALF, CH), jnp.bfloat16),
            pltpu.VMEM((QTR, CH), jnp.bfloat16),
            pltpu.SemaphoreType.DMA((8,)),
            pltpu.SemaphoreType.DMA((8,)),
            pltpu.SemaphoreType.DMA((4,)),
        ],
        compiler_params=pltpu.CompilerParams(
            collective_id=0,
            vmem_limit_bytes=60 * 1024 * 1024,
        ),
    )(p)


def _matmul_body(dy_ref, w_ref, out_ref):
    out_ref[...] = lax.dot_general(
        dy_ref[...], w_ref[...], (((1,), (1,)), ((), ())),
        preferred_element_type=jnp.float32,
    ).astype(jnp.bfloat16)


def _partial_matmul(dy_q, w_q):
    TM, TN = 512, 1024
    return pl.pallas_call(
        _matmul_body,
        grid=(D // TN, M // TM),
        in_specs=[
            pl.BlockSpec((TM, KQ), lambda n, m: (m, 0)),
            pl.BlockSpec((TN, KQ), lambda n, m: (n, 0)),
        ],
        out_specs=pl.BlockSpec((TM, TN), lambda n, m: (m, n)),
        out_shape=jax.ShapeDtypeStruct((M, D), jnp.bfloat16),
    )(dy_q, w_q)


def kernel(dy, W):
    my_y = lax.axis_index("y")
    dy_q = lax.dynamic_slice_in_dim(dy, my_y * KQ, KQ, axis=1)
    W_q = lax.dynamic_slice_in_dim(W, my_y * KQ, KQ, axis=1)
    partial = _partial_matmul(
        dy_q.astype(jnp.bfloat16), W_q.astype(jnp.bfloat16))
    return _allreduce_bf16(partial).astype(jnp.float32)
